# Initial kernel scaffold; baseline (speedup 1.0000x reference)
import functools

import jax
import jax.numpy as jnp
from jax import lax
from jax.experimental import pallas as pl
from jax.experimental.pallas import tpu as pltpu
from jax.experimental.pallas import tpu_sc as plsc

_NC = 2
_NS = 16
_LANES = 16
_CHUNK = 128
_G = 64


@functools.lru_cache(maxsize=None)
def _make_sc_agg(N, D, E):
    assert E % _CHUNK == 0 and N % _NS == 0 and D % _LANES == 0
    n_chunks = E // _CHUNK
    NW = _NC * _NS
    base = n_chunks // NW
    rem = n_chunks % NW
    cap = base + (1 if rem else 0)
    rows_per_sub = N // _NS
    full = rows_per_sub // _CHUNK
    tail = rows_per_sub % _CHUNK
    mesh = plsc.VectorSubcoreMesh(core_axis_name="c", subcore_axis_name="s",
                                  num_cores=_NC, num_subcores=_NS)

    def body(x_hbm, src_hbm, dst_hbm, out_hbm, acc_shr, src_v, dst_v, rows_v,
             sem):
        cid = lax.axis_index("c")
        sid = lax.axis_index("s")
        wid = sid * _NC + cid

        zero16 = jnp.zeros((_LANES,), jnp.float32)

        def _zrow(r, carry):
            for l in range(D // _LANES):
                rows_v[r, pl.ds(l * _LANES, _LANES)] = zero16
            return carry

        lax.fori_loop(0, _CHUNK, _zrow, 0)
        base_row = sid * rows_per_sub
        for k in range(full):
            pltpu.sync_copy(rows_v,
                            acc_shr.at[pl.ds(base_row + k * _CHUNK, _CHUNK)])
        if tail:
            pltpu.sync_copy(rows_v.at[pl.ds(0, tail)],
                            acc_shr.at[pl.ds(base_row + full * _CHUNK, tail)])

        start_c = wid * base + jnp.minimum(wid, rem)
        if base:
            pltpu.sync_copy(src_hbm.at[pl.ds(start_c, base)],
                            src_v.at[pl.ds(0, base)])
            pltpu.sync_copy(dst_hbm.at[pl.ds(start_c, base)],
                            dst_v.at[pl.ds(0, base)])
        if rem:
            @pl.when(wid < rem)
            def _():
                pltpu.sync_copy(src_hbm.at[pl.ds(start_c + base, 1)],
                                src_v.at[pl.ds(base, 1)])
                pltpu.sync_copy(dst_hbm.at[pl.ds(start_c + base, 1)],
                                dst_v.at[pl.ds(base, 1)])

        plsc.subcore_barrier()

        n_w = base + jnp.where(wid < rem, 1, 0)

        def _chunk(ci, carry):
            pltpu.async_copy(x_hbm.at[src_v.at[ci]], rows_v, sem).wait()
            pltpu.sync_copy(rows_v, acc_shr.at[dst_v.at[ci]], add=True)
            return carry

        lax.fori_loop(0, n_w, _chunk, 0)

        plsc.subcore_barrier()
        pltpu.sync_copy(acc_shr.at[pl.ds(base_row, rows_per_sub)],
                        out_hbm.at[cid, pl.ds(base_row, rows_per_sub)])

    return pl.kernel(
        body,
        out_type=jax.ShapeDtypeStruct((_NC, N, D), jnp.float32),
        mesh=mesh,
        scratch_types=[
            pltpu.VMEM_SHARED((N, D), jnp.float32),
            pltpu.VMEM((cap, _CHUNK), jnp.int32),
            pltpu.VMEM((cap, _CHUNK), jnp.int32),
            pltpu.VMEM((_CHUNK, D), jnp.float32),
            pltpu.SemaphoreType.DMA,
        ],
    )


def _tc_layer(p, x, WlT, WrT, b, BN):
    N, D = x.shape
    H = WlT.shape[1]
    NB = N // BN

    def body(p_ref, x_ref, wl_ref, wr_ref, b_ref, o_ref):
        agg = p_ref[0] + p_ref[1]
        acc = jnp.dot(agg, wl_ref[...], preferred_element_type=jnp.float32)
        acc += jnp.dot(x_ref[...], wr_ref[...],
                       preferred_element_type=jnp.float32)
        o_ref[...] = jnp.maximum(acc + b_ref[...], 0.0)

    return pl.pallas_call(
        body,
        grid=(NB,),
        in_specs=[
            pl.BlockSpec((_NC, BN, D), lambda i: (0, i, 0)),
            pl.BlockSpec((BN, D), lambda i: (i, 0)),
            pl.BlockSpec((D, H), lambda i: (0, 0)),
            pl.BlockSpec((D, H), lambda i: (0, 0)),
            pl.BlockSpec((1, H), lambda i: (0, 0)),
        ],
        out_specs=pl.BlockSpec((BN, H), lambda i: (i, 0)),
        out_shape=jax.ShapeDtypeStruct((N, H), jnp.float32),
    )(p, x, WlT, WrT, b)


def _tc_layer2_pool(p, h, WlT, WrT, b, batch3, BN):
    N, D = h.shape
    H = WlT.shape[1]
    NB = N // BN

    def body(p_ref, h_ref, wl_ref, wr_ref, b_ref, bt_ref, pooled_ref, cnt_ref):
        i = pl.program_id(0)
        agg = p_ref[0] + p_ref[1]
        acc = jnp.dot(agg, wl_ref[...], preferred_element_type=jnp.float32)
        acc += jnp.dot(h_ref[...], wr_ref[...],
                       preferred_element_type=jnp.float32)
        h2 = jnp.maximum(acc + b_ref[...], 0.0)
        bt = bt_ref[0]
        gids = lax.broadcasted_iota(jnp.int32, (_G, BN), 0)
        mask = (gids == bt).astype(jnp.float32)
        psum = jnp.dot(mask, h2, preferred_element_type=jnp.float32)
        csum = jnp.sum(mask, axis=1, keepdims=True)

        @pl.when(i == 0)
        def _():
            pooled_ref[...] = psum
            cnt_ref[...] = csum

        @pl.when(i > 0)
        def _():
            pooled_ref[...] += psum
            cnt_ref[...] += csum

    return pl.pallas_call(
        body,
        grid=(NB,),
        in_specs=[
            pl.BlockSpec((_NC, BN, D), lambda i: (0, i, 0)),
            pl.BlockSpec((BN, D), lambda i: (i, 0)),
            pl.BlockSpec((D, H), lambda i: (0, 0)),
            pl.BlockSpec((D, H), lambda i: (0, 0)),
            pl.BlockSpec((1, H), lambda i: (0, 0)),
            pl.BlockSpec((1, 1, BN), lambda i: (i, 0, 0)),
        ],
        out_specs=[
            pl.BlockSpec((_G, H), lambda i: (0, 0)),
            pl.BlockSpec((_G, 1), lambda i: (0, 0)),
        ],
        out_shape=[
            jax.ShapeDtypeStruct((_G, H), jnp.float32),
            jax.ShapeDtypeStruct((_G, 1), jnp.float32),
        ],
    )(p, h, WlT, WrT, b, batch3)


def _tc_head(pooled, cnt, WfcT, bfc):
    O = WfcT.shape[1]

    def body(pooled_ref, cnt_ref, wfc_ref, bfc_ref, o_ref):
        mean = pooled_ref[...] / jnp.maximum(cnt_ref[...], 1.0)
        logits = jnp.dot(mean, wfc_ref[...],
                         preferred_element_type=jnp.float32) + bfc_ref[...]
        m = jnp.max(logits, axis=1, keepdims=True)
        shifted = logits - m
        lse = jnp.log(jnp.sum(jnp.exp(shifted), axis=1, keepdims=True))
        o_ref[...] = shifted - lse

    return pl.pallas_call(
        body,
        out_shape=jax.ShapeDtypeStruct((_G, O), jnp.float32),
    )(pooled, cnt, WfcT, bfc)


def kernel(x, edge_index, batch, W1l, b1l, W1r, W2l, b2l, W2r, Wfc, bfc):
    N, D = x.shape
    E = edge_index.shape[1]
    BN = 1000
    src = edge_index[0].reshape(E // _CHUNK, _CHUNK)
    dst = edge_index[1].reshape(E // _CHUNK, _CHUNK)
    batch3 = batch.reshape(N // BN, 1, BN)

    sc_agg = _make_sc_agg(N, D, E)
    p1 = sc_agg(x, src, dst)
    h1 = _tc_layer(p1, x, W1l.T, W1r.T, b1l.reshape(1, -1), BN)
    p2 = sc_agg(h1, src, dst)
    pooled, cnt = _tc_layer2_pool(p2, h1, W2l.T, W2r.T, b2l.reshape(1, -1),
                                  batch3, BN)
    return _tc_head(pooled, cnt, Wfc.T, bfc.reshape(1, -1))


# speedup vs baseline: 3.3392x; 3.3392x over previous
import functools

import jax
import jax.numpy as jnp
from jax import lax
from jax.experimental import pallas as pl
from jax.experimental.pallas import tpu as pltpu
from jax.experimental.pallas import tpu_sc as plsc

_NC = 2
_NS = 16
_LANES = 16
_CHUNK = 128
_G = 64


@functools.lru_cache(maxsize=None)
def _make_sc_agg(N_acc, D, E_pad):
    NW = _NC * _NS
    assert E_pad % (_CHUNK * NW) == 0 and N_acc % _NS == 0 and D % _LANES == 0
    n_chunks = E_pad // _CHUNK
    base = n_chunks // NW
    rows_per_sub = N_acc // _NS
    full = rows_per_sub // _CHUNK
    tail = rows_per_sub % _CHUNK
    mesh = plsc.VectorSubcoreMesh(core_axis_name="c", subcore_axis_name="s",
                                  num_cores=_NC, num_subcores=_NS)

    def body(x_hbm, src_hbm, dst_hbm, out_hbm, acc_shr, src_v, dst_v, rows_v,
             sem):
        cid = lax.axis_index("c")
        sid = lax.axis_index("s")
        wid = sid * _NC + cid

        zero16 = jnp.zeros((_LANES,), jnp.float32)

        def _zrow(r, carry):
            for l in range(D // _LANES):
                rows_v[r, pl.ds(l * _LANES, _LANES)] = zero16
            return carry

        lax.fori_loop(0, _CHUNK, _zrow, 0)
        base_row = sid * rows_per_sub
        for k in range(full):
            pltpu.sync_copy(rows_v,
                            acc_shr.at[pl.ds(base_row + k * _CHUNK, _CHUNK)])
        if tail:
            pltpu.sync_copy(rows_v.at[pl.ds(0, tail)],
                            acc_shr.at[pl.ds(base_row + full * _CHUNK, tail)])

        start_c = wid * base
        pltpu.sync_copy(src_hbm.at[pl.ds(start_c, base)], src_v)
        pltpu.sync_copy(dst_hbm.at[pl.ds(start_c, base)], dst_v)

        plsc.subcore_barrier()

        def _chunk(ci, carry):
            pltpu.async_copy(x_hbm.at[src_v.at[ci]], rows_v, sem).wait()
            pltpu.sync_copy(rows_v, acc_shr.at[dst_v.at[ci]], add=True)
            return carry

        lax.fori_loop(0, base, _chunk, 0)

        plsc.subcore_barrier()
        pltpu.sync_copy(acc_shr.at[pl.ds(base_row, rows_per_sub)],
                        out_hbm.at[cid, pl.ds(base_row, rows_per_sub)])

    return pl.kernel(
        body,
        out_type=jax.ShapeDtypeStruct((_NC, N_acc, D), jnp.float32),
        mesh=mesh,
        scratch_types=[
            pltpu.VMEM_SHARED((N_acc, D), jnp.float32),
            pltpu.VMEM((base, _CHUNK), jnp.int32),
            pltpu.VMEM((base, _CHUNK), jnp.int32),
            pltpu.VMEM((_CHUNK, D), jnp.float32),
            pltpu.SemaphoreType.DMA,
        ],
    )


def _tc_layer(p, x, WlT, WrT, b, BN):
    N, D = x.shape
    H = WlT.shape[1]
    NB = N // BN

    def body(p_ref, x_ref, wl_ref, wr_ref, b_ref, o_ref):
        agg = p_ref[0] + p_ref[1]
        acc = jnp.dot(agg, wl_ref[...], preferred_element_type=jnp.float32)
        acc += jnp.dot(x_ref[...], wr_ref[...],
                       preferred_element_type=jnp.float32)
        o_ref[...] = jnp.maximum(acc + b_ref[...], 0.0)

    return pl.pallas_call(
        body,
        grid=(NB,),
        in_specs=[
            pl.BlockSpec((_NC, BN, D), lambda i: (0, i, 0)),
            pl.BlockSpec((BN, D), lambda i: (i, 0)),
            pl.BlockSpec((D, H), lambda i: (0, 0)),
            pl.BlockSpec((D, H), lambda i: (0, 0)),
            pl.BlockSpec((1, H), lambda i: (0, 0)),
        ],
        out_specs=pl.BlockSpec((BN, H), lambda i: (i, 0)),
        out_shape=jax.ShapeDtypeStruct((N, H), jnp.float32),
    )(p, x, WlT, WrT, b)


def _tc_layer2_pool(p, h, WlT, WrT, b, batch3, BN):
    N, D = h.shape
    H = WlT.shape[1]
    NB = N // BN

    def body(p_ref, h_ref, wl_ref, wr_ref, b_ref, bt_ref, pooled_ref, cnt_ref):
        i = pl.program_id(0)
        agg = p_ref[0] + p_ref[1]
        acc = jnp.dot(agg, wl_ref[...], preferred_element_type=jnp.float32)
        acc += jnp.dot(h_ref[...], wr_ref[...],
                       preferred_element_type=jnp.float32)
        h2 = jnp.maximum(acc + b_ref[...], 0.0)
        bt = bt_ref[0]
        gids = lax.broadcasted_iota(jnp.int32, (_G, BN), 0)
        mask = (gids == bt).astype(jnp.float32)
        psum = jnp.dot(mask, h2, preferred_element_type=jnp.float32)
        csum = jnp.sum(mask, axis=1, keepdims=True)

        @pl.when(i == 0)
        def _():
            pooled_ref[...] = psum
            cnt_ref[...] = csum

        @pl.when(i > 0)
        def _():
            pooled_ref[...] += psum
            cnt_ref[...] += csum

    return pl.pallas_call(
        body,
        grid=(NB,),
        in_specs=[
            pl.BlockSpec((_NC, BN, D), lambda i: (0, i, 0)),
            pl.BlockSpec((BN, D), lambda i: (i, 0)),
            pl.BlockSpec((D, H), lambda i: (0, 0)),
            pl.BlockSpec((D, H), lambda i: (0, 0)),
            pl.BlockSpec((1, H), lambda i: (0, 0)),
            pl.BlockSpec((1, 1, BN), lambda i: (i, 0, 0)),
        ],
        out_specs=[
            pl.BlockSpec((_G, H), lambda i: (0, 0)),
            pl.BlockSpec((_G, 1), lambda i: (0, 0)),
        ],
        out_shape=[
            jax.ShapeDtypeStruct((_G, H), jnp.float32),
            jax.ShapeDtypeStruct((_G, 1), jnp.float32),
        ],
    )(p, h, WlT, WrT, b, batch3)


def _tc_head(pooled, cnt, WfcT, bfc):
    O = WfcT.shape[1]

    def body(pooled_ref, cnt_ref, wfc_ref, bfc_ref, o_ref):
        mean = pooled_ref[...] / jnp.maximum(cnt_ref[...], 1.0)
        logits = jnp.dot(mean, wfc_ref[...],
                         preferred_element_type=jnp.float32) + bfc_ref[...]
        m = jnp.max(logits, axis=1, keepdims=True)
        shifted = logits - m
        lse = jnp.log(jnp.sum(jnp.exp(shifted), axis=1, keepdims=True))
        o_ref[...] = shifted - lse

    return pl.pallas_call(
        body,
        out_shape=jax.ShapeDtypeStruct((_G, O), jnp.float32),
    )(pooled, cnt, WfcT, bfc)


def kernel(x, edge_index, batch, W1l, b1l, W1r, W2l, b2l, W2r, Wfc, bfc):
    N, D = x.shape
    E = edge_index.shape[1]
    BN = 1000
    NW = _NC * _NS
    quantum = _CHUNK * NW * 8
    E_pad = -(-E // quantum) * quantum
    N_acc = -(-(N + 1) // 128) * 128
    pad = E_pad - E
    src = jnp.concatenate([edge_index[0],
                           jnp.zeros((pad,), jnp.int32)]).reshape(-1, _CHUNK)
    dst = jnp.concatenate([edge_index[1],
                           jnp.full((pad,), N, jnp.int32)]).reshape(-1, _CHUNK)
    batch3 = batch.reshape(N // BN, 1, BN)

    sc_agg = _make_sc_agg(N_acc, D, E_pad)
    p1 = sc_agg(x, src, dst)
    h1 = _tc_layer(p1, x, W1l.T, W1r.T, b1l.reshape(1, -1), BN)
    p2 = sc_agg(h1, src, dst)
    pooled, cnt = _tc_layer2_pool(p2, h1, W2l.T, W2r.T, b2l.reshape(1, -1),
                                  batch3, BN)
    return _tc_head(pooled, cnt, Wfc.T, bfc.reshape(1, -1))
